# baseline (speedup 1.0000x reference)
import jax
import jax.numpy as jnp
from jax.experimental import pallas as pl
from jax.experimental.pallas import tpu as pltpu

_B, _C, _H, _W = 16, 64, 32, 32
_HW = _H * _W
_K = 512


_MB = 8


def _vq_body(z_ref, cb_ref, zq_ref, idx_ref):
    cb = cb_ref[...]
    esq = jnp.sum(cb * cb, axis=1, keepdims=True)
    kio = jax.lax.broadcasted_iota(jnp.int32, (_K, _HW), 0)
    for i in range(_MB):
        zb = z_ref[i]
        dot = jax.lax.dot_general(
            cb, zb, (((1,), (0,)), ((), ())),
            preferred_element_type=jnp.float32,
        )
        s = zb * zb
        w = _C
        while w > 1:
            w //= 2
            s = s[:w] + s[w:2 * w]
        zsq = s
        d = zsq - 2.0 * dot + esq
        dmin = jnp.min(d, axis=0, keepdims=True)
        idx = jnp.min(jnp.where(d == dmin, kio, _K), axis=0).astype(jnp.int32)
        oh = (kio == idx[None, :]).astype(jnp.float32)
        zq = jax.lax.dot_general(
            cb, oh, (((0,), (0,)), ((), ())),
            preferred_element_type=jnp.float32,
        )
        zq_ref[i] = zq
        idx_ref[i] = idx.reshape(8, 128)


def kernel(z_e, codebook):
    B, C, H, W = z_e.shape
    z = z_e.reshape(B, C, H * W)
    zq, idx = pl.pallas_call(
        _vq_body,
        grid=(B // _MB,),
        in_specs=[
            pl.BlockSpec((_MB, C, H * W), lambda b: (b, 0, 0)),
            pl.BlockSpec((_K, C), lambda b: (0, 0)),
        ],
        out_specs=[
            pl.BlockSpec((_MB, C, H * W), lambda b: (b, 0, 0)),
            pl.BlockSpec((_MB, 8, 128), lambda b: (b, 0, 0)),
        ],
        out_shape=[
            jax.ShapeDtypeStruct((B, C, H * W), jnp.float32),
            jax.ShapeDtypeStruct((B, 8, 128), jnp.int32),
        ],
        compiler_params=pltpu.CompilerParams(
            dimension_semantics=("arbitrary",),
        ),
    )(z, codebook)
    return zq.reshape(B, C, H, W), idx.reshape(-1)
